# Initial kernel scaffold; baseline (speedup 1.0000x reference)
import jax
import jax.numpy as jnp
from jax.experimental import pallas as pl


def kernel(attention_weights_1, attention_weights_2):
    raise NotImplementedError("write your pallas kernel here")


# speedup vs baseline: 3.7249x; 3.7249x over previous
import jax
import jax.numpy as jnp
from jax.experimental import pallas as pl
from jax.experimental.pallas import tpu as pltpu

B = 128
S = 32768
K = 5
RB = 8
GRID = B // RB


def _block_stats(a):
    s = jnp.sum(a, axis=1)
    q = jnp.sum(a * a, axis=1)
    m = s / S
    v = (q - S * m * m) / (S - 1)
    return m, v


def _topk_hist(a, colid, hist_ref):
    acc = jnp.zeros(a.shape, jnp.float32)
    x = a
    for _ in range(K):
        m = jnp.max(x, axis=1, keepdims=True)
        cand = jnp.where(x == m, colid, S)
        idx = jnp.min(cand, axis=1, keepdims=True)
        hit = colid == idx
        acc += hit.astype(jnp.float32)
        x = jnp.where(hit, -1.0, x)
    hist_ref[...] += jnp.sum(acc, axis=0, keepdims=True)


def _body(a1_ref, a2_ref, out_ref, hist1_ref, hist2_ref, stats_ref):
    i = pl.program_id(0)

    @pl.when(i == 0)
    def _init():
        hist1_ref[...] = jnp.zeros((1, S), jnp.float32)
        hist2_ref[...] = jnp.zeros((1, S), jnp.float32)
        stats_ref[0] = 0.0
        stats_ref[1] = 0.0

    a1 = a1_ref[...]
    a2 = a2_ref[...]

    m1, v1 = _block_stats(a1)
    m2, v2 = _block_stats(a2)
    dm = m1 - m2
    dv = v1 - v2
    stats_ref[0] += jnp.sum(dm * dm)
    stats_ref[1] += jnp.sum(dv * dv)

    colid = jax.lax.broadcasted_iota(jnp.int32, (RB, S), 1)
    _topk_hist(a1, colid, hist1_ref)
    _topk_hist(a2, colid, hist2_ref)

    @pl.when(i == GRID - 1)
    def _finish():
        p1 = hist1_ref[...] * (1.0 / B)
        p2 = hist2_ref[...] * (1.0 / B)
        e1 = jnp.exp(p1)
        e2 = jnp.exp(p2)
        se1 = jnp.sum(e1)
        se2 = jnp.sum(e2)
        t = jnp.sum(e2 * (p2 - p1)) / se2
        corr = (t + jnp.log(se1) - jnp.log(se2)) / S
        dist = stats_ref[0] / B + stats_ref[1] / B
        out_ref[0] = dist + corr
        out_ref[1] = dist
        out_ref[2] = corr


def kernel(attention_weights_1, attention_weights_2):
    a1 = attention_weights_1.reshape(B, S)
    a2 = attention_weights_2.reshape(B, S)
    out = pl.pallas_call(
        _body,
        grid=(GRID,),
        in_specs=[
            pl.BlockSpec((RB, S), lambda i: (i, 0)),
            pl.BlockSpec((RB, S), lambda i: (i, 0)),
        ],
        out_specs=pl.BlockSpec(memory_space=pltpu.SMEM),
        out_shape=jax.ShapeDtypeStruct((3,), jnp.float32),
        scratch_shapes=[
            pltpu.VMEM((1, S), jnp.float32),
            pltpu.VMEM((1, S), jnp.float32),
            pltpu.SMEM((2,), jnp.float32),
        ],
        compiler_params=pltpu.CompilerParams(
            dimension_semantics=("arbitrary",),
        ),
    )(a1, a2)
    return (out[0], out[1], out[2])


# speedup vs baseline: 4.1538x; 1.1151x over previous
import jax
import jax.numpy as jnp
from jax.experimental import pallas as pl
from jax.experimental.pallas import tpu as pltpu

B = 128
S = 32768
K = 5
RB = 8
GRID = B // RB


def _block_stats(a):
    s = jnp.sum(a, axis=1)
    q = jnp.sum(a * a, axis=1)
    m = s / S
    v = (q - S * m * m) / (S - 1)
    return m, v


def _topk_hist(a, colid, hist_ref):
    x = a
    for _ in range(K):
        m = jnp.max(x, axis=1, keepdims=True)
        cand = jnp.where(x == m, colid, S)
        idx = jnp.min(cand, axis=1, keepdims=True)
        x = jnp.where(colid == idx, -1.0, x)
    hist_ref[...] += jnp.sum((x < 0.0).astype(jnp.float32), axis=0, keepdims=True)


def _body(a1_ref, a2_ref, out_ref, hist1_ref, hist2_ref, stats_ref):
    i = pl.program_id(0)

    @pl.when(i == 0)
    def _init():
        hist1_ref[...] = jnp.zeros((1, S), jnp.float32)
        hist2_ref[...] = jnp.zeros((1, S), jnp.float32)
        stats_ref[0] = 0.0
        stats_ref[1] = 0.0

    a1 = a1_ref[...]
    a2 = a2_ref[...]

    m1, v1 = _block_stats(a1)
    m2, v2 = _block_stats(a2)
    dm = m1 - m2
    dv = v1 - v2
    stats_ref[0] += jnp.sum(dm * dm)
    stats_ref[1] += jnp.sum(dv * dv)

    colid = jax.lax.broadcasted_iota(jnp.int32, (RB, S), 1)
    _topk_hist(a1, colid, hist1_ref)
    _topk_hist(a2, colid, hist2_ref)

    @pl.when(i == GRID - 1)
    def _finish():
        p1 = hist1_ref[...] * (1.0 / B)
        p2 = hist2_ref[...] * (1.0 / B)
        e1 = jnp.exp(p1)
        e2 = jnp.exp(p2)
        se1 = jnp.sum(e1)
        se2 = jnp.sum(e2)
        t = jnp.sum(e2 * (p2 - p1)) / se2
        corr = (t + jnp.log(se1) - jnp.log(se2)) / S
        dist = stats_ref[0] / B + stats_ref[1] / B
        out_ref[0] = dist + corr
        out_ref[1] = dist
        out_ref[2] = corr


def kernel(attention_weights_1, attention_weights_2):
    a1 = attention_weights_1.reshape(B, S)
    a2 = attention_weights_2.reshape(B, S)
    out = pl.pallas_call(
        _body,
        grid=(GRID,),
        in_specs=[
            pl.BlockSpec((RB, S), lambda i: (i, 0)),
            pl.BlockSpec((RB, S), lambda i: (i, 0)),
        ],
        out_specs=pl.BlockSpec(memory_space=pltpu.SMEM),
        out_shape=jax.ShapeDtypeStruct((3,), jnp.float32),
        scratch_shapes=[
            pltpu.VMEM((1, S), jnp.float32),
            pltpu.VMEM((1, S), jnp.float32),
            pltpu.SMEM((2,), jnp.float32),
        ],
        compiler_params=pltpu.CompilerParams(
            dimension_semantics=("arbitrary",),
        ),
    )(a1, a2)
    return (out[0], out[1], out[2])
